# Initial kernel scaffold; baseline (speedup 1.0000x reference)
import jax
import jax.numpy as jnp
from jax.experimental import pallas as pl


def kernel(D, len_a, len_b):
    raise NotImplementedError("write your pallas kernel here")


# speedup vs baseline: 36.7029x; 36.7029x over previous
import jax
import jax.numpy as jnp
from jax.experimental import pallas as pl
from jax.experimental.pallas import tpu as pltpu

GAMMA = 1.0
LBD_ORD = 1.0
LBD_DUR = 5.0
LBD_LEN = 0.2
SIGMA = 1.0
MARGIN = 1.0

SUB = 8
LANES = 640


def _dp_kernel(ns_ref, scal_ref, dpad_ref, bld_ref, kp_ref, ku_ref, out_ref):
    lanes = jax.lax.broadcasted_iota(jnp.int32, (SUB, LANES), 1).astype(jnp.float32)
    scal = scal_ref[...]
    Mf = scal[:, 0:1]
    Nf = scal[:, 1:2]
    wsf = scal[:, 2:3]
    lds = scal[:, 3:4]
    ldo = scal[:, 4:5]
    inf = jnp.float32(jnp.inf)

    kp = kp_ref[...]
    ku = ku_ref[...]
    prev0 = jnp.where(lanes == 0.0, 0.0, inf)
    t0 = jnp.zeros((SUB, LANES), jnp.float32)

    def body(i, carry):
        prev, tacc = carry
        fi = i.astype(jnp.float32)
        curr_lb = jnp.maximum(1.0, fi - wsf)
        curr_rb = jnp.minimum(Nf + 2.0, fi + wsf)
        prev_lb = jnp.maximum(0.0, curr_lb - 1.0)
        prev_rb = jnp.minimum(Nf + 2.0, curr_rb - 1.0)
        pmask = (lanes >= prev_lb) & (lanes < prev_rb)
        band = (lanes >= curr_lb) & (lanes < curr_rb)

        m = jnp.min(jnp.where(pmask, prev, inf), axis=1, keepdims=True)
        u = prev + lanes
        mu = jnp.min(jnp.where(pmask, u, inf), axis=1, keepdims=True)
        ep = jnp.where(pmask, jnp.exp(m - prev), 0.0)
        eu = jnp.where(pmask, jnp.exp(mu - u), 0.0)
        sp = jnp.dot(ep, kp, preferred_element_type=jnp.float32)
        su = jnp.dot(eu, ku, preferred_element_type=jnp.float32)
        softmin = jnp.minimum(m - jnp.log(sp), mu - lanes - jnp.log(su))

        drow = dpad_ref[i]
        ldrow = bld_ref[i]
        vals = drow + (ldrow * lds + ldo) + softmin
        row = jnp.where(band, vals, inf)
        new_prev = jnp.where(fi < Mf + 2.0, row, prev)
        hit = (fi == Mf) & (lanes == Nf)
        tacc = tacc + jnp.where(hit, row, 0.0)
        return new_prev, tacc

    _, tacc = jax.lax.fori_loop(1, ns_ref[0], body, (prev0, t0))
    out_ref[...] = jnp.broadcast_to(
        jnp.sum(tacc, axis=1, keepdims=True), (SUB, 128)
    )


@jax.jit
def kernel(D, len_a, len_b):
    B, L1, L2 = D.shape

    Mf = len_a.astype(jnp.float32)
    Nf = len_b.astype(jnp.float32)
    dM = jnp.abs(Mf - Nf)
    wsf = jnp.maximum(5.0, dM + 1.0)
    lds = LBD_DUR / Mf
    ldo = dM * LBD_LEN / Mf
    cols = jnp.stack([Mf, Nf, wsf, lds, ldo], axis=1)
    scal = jnp.zeros((SUB, 128), jnp.float32)
    scal = scal.at[:B, :5].set(cols)
    scal = scal.at[B:, 0].set(256.0)
    scal = scal.at[B:, 1].set(256.0)
    scal = scal.at[B:, 2].set(5.0)
    scal = scal.at[B:, 3].set(LBD_DUR / 256.0)

    nsteps = jnp.max(len_a).astype(jnp.int32) + 1

    Dr = jnp.concatenate([D, D[:, -1:, :]], axis=1)
    Dt = jnp.transpose(Dr, (1, 0, 2))
    dpad = jnp.zeros((L1 + 2, SUB, LANES), jnp.float32)
    dpad = dpad.at[1:, :B, 1 : L2 + 1].set(Dt)

    ii = jnp.arange(L1 + 2, dtype=jnp.float32)[:, None]
    jj = jnp.arange(LANES, dtype=jnp.float32)[None, :]
    bld = 1.0 - jnp.exp(-((ii - jj) ** 2) / (2.0 * SIGMA**2 * (jj + 1.0)))
    bld = bld.reshape(L1 + 2, 1, LANES)

    jp = jnp.arange(LANES, dtype=jnp.float32)[:, None]
    jc = jnp.arange(LANES, dtype=jnp.float32)[None, :]
    kp = jnp.where(jp <= jc - 1.0, 1.0, jnp.exp(-(jp - jc + 1.0)))
    ku = jnp.where(jp <= jc - 1.0, jnp.exp(-(jc - jp)), jnp.exp(-1.0))

    out = pl.pallas_call(
        _dp_kernel,
        out_shape=jax.ShapeDtypeStruct((SUB, 128), jnp.float32),
        in_specs=[
            pl.BlockSpec(memory_space=pltpu.SMEM),
            pl.BlockSpec(memory_space=pltpu.VMEM),
            pl.BlockSpec(memory_space=pltpu.VMEM),
            pl.BlockSpec(memory_space=pltpu.VMEM),
            pl.BlockSpec(memory_space=pltpu.VMEM),
            pl.BlockSpec(memory_space=pltpu.VMEM),
        ],
    )(
        nsteps.reshape(1),
        scal,
        dpad,
        bld,
        kp.astype(jnp.float32),
        ku.astype(jnp.float32),
    )
    return out[:B, 0]


# speedup vs baseline: 46.1883x; 1.2584x over previous
import jax
import jax.numpy as jnp
from jax.experimental import pallas as pl
from jax.experimental.pallas import tpu as pltpu

GAMMA = 1.0
LBD_ORD = 1.0
LBD_DUR = 5.0
LBD_LEN = 0.2
SIGMA = 1.0
MARGIN = 1.0

SUB = 8
LANES = 640
BLK = 128
NBLK = LANES // BLK


def _dp_kernel(ns_ref, scal_ref, dpad_ref, bld_ref, kps_ref, kus_ref, out_ref):
    lanes = jax.lax.broadcasted_iota(jnp.int32, (SUB, LANES), 1).astype(jnp.float32)
    scal = scal_ref[...]
    Mf = scal[:, 0:1]
    Nf = scal[:, 1:2]
    wsf = scal[:, 2:3]
    lds = scal[:, 3:4]
    ldo = scal[:, 4:5]
    inf = jnp.float32(jnp.inf)
    einv = jnp.float32(0.36787944117144233)

    kps = kps_ref[...]
    kus = kus_ref[...]
    prev0 = jnp.where(lanes == 0.0, 0.0, inf)
    t0 = jnp.zeros((SUB, LANES), jnp.float32)

    def body(i, carry):
        prev, tacc = carry
        fi = i.astype(jnp.float32)
        curr_lb = jnp.maximum(1.0, fi - wsf)
        curr_rb = jnp.minimum(Nf + 2.0, fi + wsf)
        prev_lb = jnp.maximum(0.0, curr_lb - 1.0)
        prev_rb = jnp.minimum(Nf + 2.0, curr_rb - 1.0)
        pmask = (lanes >= prev_lb) & (lanes < prev_rb)
        band = (lanes >= curr_lb) & (lanes < curr_rb)

        m = jnp.min(jnp.where(pmask, prev, inf), axis=1, keepdims=True)
        u = prev + lanes
        mu = jnp.min(jnp.where(pmask, u, inf), axis=1, keepdims=True)
        ep = jnp.where(pmask, jnp.exp(m - prev), 0.0)
        eu = jnp.where(pmask, jnp.exp(mu - u), 0.0)
        epb = ep.astype(jnp.bfloat16)
        eub = eu.astype(jnp.bfloat16)

        zblk = jnp.zeros((SUB, BLK), jnp.bfloat16)
        ep_blocks = [epb[:, b * BLK : (b + 1) * BLK] for b in range(NBLK)] + [zblk]
        eu_blocks = [zblk] + [eub[:, b * BLK : (b + 1) * BLK] for b in range(NBLK)]
        tp = [jnp.sum(ep[:, b * BLK : (b + 1) * BLK], axis=1, keepdims=True)
              for b in range(NBLK)]
        tu = [jnp.sum(eu[:, b * BLK : (b + 1) * BLK], axis=1, keepdims=True)
              for b in range(NBLK)]

        sp_blocks = []
        su_blocks = []
        cpre = jnp.zeros((SUB, 1), jnp.float32)
        for b in range(NBLK):
            inp = jnp.concatenate([ep_blocks[b], ep_blocks[b + 1]], axis=1)
            sp_blocks.append(
                jnp.dot(inp, kps, preferred_element_type=jnp.float32) + cpre
            )
            cpre = cpre + tp[b]
        csuf = jnp.zeros((SUB, 1), jnp.float32)
        for b in range(NBLK - 1, -1, -1):
            inp = jnp.concatenate([eu_blocks[b], eu_blocks[b + 1]], axis=1)
            su_blocks.append(
                jnp.dot(inp, kus, preferred_element_type=jnp.float32)
                + einv * csuf
            )
            csuf = csuf + tu[b]
        sp = jnp.concatenate(sp_blocks, axis=1)
        su = jnp.concatenate(su_blocks[::-1], axis=1)

        softmin = jnp.minimum(m - jnp.log(sp), mu - lanes - jnp.log(su))

        drow = dpad_ref[i]
        ldrow = bld_ref[i]
        vals = drow + (ldrow * lds + ldo) + softmin
        row = jnp.where(band, vals, inf)
        new_prev = jnp.where(fi < Mf + 2.0, row, prev)
        hit = (fi == Mf) & (lanes == Nf)
        tacc = tacc + jnp.where(hit, row, 0.0)
        return new_prev, tacc

    _, tacc = jax.lax.fori_loop(1, ns_ref[0], body, (prev0, t0))
    out_ref[...] = jnp.broadcast_to(
        jnp.sum(tacc, axis=1, keepdims=True), (SUB, 128)
    )


@jax.jit
def kernel(D, len_a, len_b):
    B, L1, L2 = D.shape

    Mf = len_a.astype(jnp.float32)
    Nf = len_b.astype(jnp.float32)
    dM = jnp.abs(Mf - Nf)
    wsf = jnp.maximum(5.0, dM + 1.0)
    lds = LBD_DUR / Mf
    ldo = dM * LBD_LEN / Mf
    cols = jnp.stack([Mf, Nf, wsf, lds, ldo], axis=1)
    scal = jnp.zeros((SUB, 128), jnp.float32)
    scal = scal.at[:B, :5].set(cols)
    scal = scal.at[B:, 0].set(256.0)
    scal = scal.at[B:, 1].set(256.0)
    scal = scal.at[B:, 2].set(5.0)
    scal = scal.at[B:, 3].set(LBD_DUR / 256.0)

    nsteps = jnp.max(len_a).astype(jnp.int32) + 1

    Dr = jnp.concatenate([D, D[:, -1:, :]], axis=1)
    Dt = jnp.transpose(Dr, (1, 0, 2))
    dpad = jnp.zeros((L1 + 2, SUB, LANES), jnp.float32)
    dpad = dpad.at[1:, :B, 1 : L2 + 1].set(Dt)

    ii = jnp.arange(L1 + 2, dtype=jnp.float32)[:, None]
    jj = jnp.arange(LANES, dtype=jnp.float32)[None, :]
    bld = 1.0 - jnp.exp(-((ii - jj) ** 2) / (2.0 * SIGMA**2 * (jj + 1.0)))
    bld = bld.reshape(L1 + 2, 1, LANES)

    t = jnp.arange(BLK, dtype=jnp.float32)[:, None]
    s = jnp.arange(BLK, dtype=jnp.float32)[None, :]
    kp_in = jnp.where(t <= s - 1.0, 1.0, jnp.exp(-(t - s + 1.0)))
    kp_nx = jnp.exp(-(BLK + t - s + 1.0))
    ku_in = jnp.where(t <= s - 1.0, jnp.exp(-(s - t)), jnp.exp(-1.0))
    ku_pv = jnp.exp(-(s + BLK - t))
    kps = jnp.concatenate([kp_in, kp_nx], axis=0).astype(jnp.bfloat16)
    kus = jnp.concatenate([ku_pv, ku_in], axis=0).astype(jnp.bfloat16)

    out = pl.pallas_call(
        _dp_kernel,
        out_shape=jax.ShapeDtypeStruct((SUB, 128), jnp.float32),
        in_specs=[
            pl.BlockSpec(memory_space=pltpu.SMEM),
            pl.BlockSpec(memory_space=pltpu.VMEM),
            pl.BlockSpec(memory_space=pltpu.VMEM),
            pl.BlockSpec(memory_space=pltpu.VMEM),
            pl.BlockSpec(memory_space=pltpu.VMEM),
            pl.BlockSpec(memory_space=pltpu.VMEM),
        ],
    )(nsteps.reshape(1), scal, dpad, bld, kps, kus)
    return out[:B, 0]


# speedup vs baseline: 71.8007x; 1.5545x over previous
import jax
import jax.numpy as jnp
from jax.experimental import pallas as pl
from jax.experimental.pallas import tpu as pltpu

GAMMA = 1.0
LBD_ORD = 1.0
LBD_DUR = 5.0
LBD_LEN = 0.2
SIGMA = 1.0
MARGIN = 1.0

SUB = 8
LANES = 640
BLK = 128
NBLK = LANES // BLK


def _dp_kernel(ns_ref, scal_ref, dpad_ref, bld_ref, kps_ref, kus_ref, out_ref):
    lanes = jax.lax.broadcasted_iota(jnp.int32, (SUB, LANES), 1).astype(jnp.float32)
    scal = scal_ref[...]
    Mf = scal[:, 0:1]
    Nf = scal[:, 1:2]
    wsf = scal[:, 2:3]
    lds = scal[:, 3:4]
    ldo = scal[:, 4:5]
    inf = jnp.float32(jnp.inf)
    einv = jnp.float32(0.36787944117144233)

    kps = kps_ref[...]
    kus = kus_ref[...]
    prev0 = jnp.where(lanes == 0.0, 0.0, inf)
    t0 = jnp.zeros((SUB, LANES), jnp.float32)

    def body(i, carry):
        prev, tacc = carry
        fi = i.astype(jnp.float32)
        curr_lb = jnp.maximum(1.0, fi - wsf)
        curr_rb = jnp.minimum(Nf + 2.0, fi + wsf)
        prev_lb = jnp.maximum(0.0, curr_lb - 1.0)
        prev_rb = jnp.minimum(Nf + 2.0, curr_rb - 1.0)
        pmask = (lanes >= prev_lb) & (lanes < prev_rb)
        band = (lanes >= curr_lb) & (lanes < curr_rb)

        m = jnp.min(jnp.where(pmask, prev, inf), axis=1, keepdims=True)
        u = prev + lanes
        mu = jnp.min(jnp.where(pmask, u, inf), axis=1, keepdims=True)
        ep = jnp.where(pmask, jnp.exp(m - prev), 0.0)
        eu = jnp.where(pmask, jnp.exp(mu - u), 0.0)
        epb = ep.astype(jnp.bfloat16)
        eub = eu.astype(jnp.bfloat16)

        zblk = jnp.zeros((SUB, BLK), jnp.bfloat16)
        ep_blocks = [epb[:, b * BLK : (b + 1) * BLK] for b in range(NBLK)] + [zblk]
        eu_blocks = [zblk] + [eub[:, b * BLK : (b + 1) * BLK] for b in range(NBLK)]
        tp = [jnp.sum(ep[:, b * BLK : (b + 1) * BLK], axis=1, keepdims=True)
              for b in range(NBLK)]
        tu = [jnp.sum(eu[:, b * BLK : (b + 1) * BLK], axis=1, keepdims=True)
              for b in range(NBLK)]

        p_in = jnp.concatenate(
            [jnp.concatenate([ep_blocks[b], ep_blocks[b + 1]], axis=1)
             for b in range(NBLK)], axis=0)
        u_in = jnp.concatenate(
            [jnp.concatenate([eu_blocks[b], eu_blocks[b + 1]], axis=1)
             for b in range(NBLK)], axis=0)
        sp_all = jnp.dot(p_in, kps, preferred_element_type=jnp.float32)
        su_all = jnp.dot(u_in, kus, preferred_element_type=jnp.float32)

        sp_blocks = []
        su_blocks = []
        cpre = jnp.zeros((SUB, 1), jnp.float32)
        csuf = jnp.zeros((SUB, 1), jnp.float32)
        for b in range(NBLK):
            sp_blocks.append(sp_all[b * SUB : (b + 1) * SUB] + cpre)
            cpre = cpre + tp[b]
        for b in range(NBLK - 1, -1, -1):
            su_blocks.append(su_all[b * SUB : (b + 1) * SUB] + einv * csuf)
            csuf = csuf + tu[b]
        sp = jnp.concatenate(sp_blocks, axis=1)
        su = jnp.concatenate(su_blocks[::-1], axis=1)

        softmin = jnp.minimum(m - jnp.log(sp), mu - lanes - jnp.log(su))

        drow = dpad_ref[i]
        ldrow = bld_ref[i]
        vals = drow + (ldrow * lds + ldo) + softmin
        row = jnp.where(band, vals, inf)
        new_prev = jnp.where(fi < Mf + 2.0, row, prev)
        hit = (fi == Mf) & (lanes == Nf)
        tacc = tacc + jnp.where(hit, row, 0.0)
        return new_prev, tacc

    def body8(idx, carry):
        for r in range(8):
            carry = body(8 * idx + 1 + r, carry)
        return carry

    _, tacc = jax.lax.fori_loop(0, ns_ref[0], body8, (prev0, t0))
    out_ref[...] = jnp.broadcast_to(
        jnp.sum(tacc, axis=1, keepdims=True), (SUB, 128)
    )


@jax.jit
def kernel(D, len_a, len_b):
    B, L1, L2 = D.shape

    Mf = len_a.astype(jnp.float32)
    Nf = len_b.astype(jnp.float32)
    dM = jnp.abs(Mf - Nf)
    wsf = jnp.maximum(5.0, dM + 1.0)
    lds = LBD_DUR / Mf
    ldo = dM * LBD_LEN / Mf
    cols = jnp.stack([Mf, Nf, wsf, lds, ldo], axis=1)
    scal = jnp.zeros((SUB, 128), jnp.float32)
    scal = scal.at[:B, :5].set(cols)
    scal = scal.at[B:, 0].set(256.0)
    scal = scal.at[B:, 1].set(256.0)
    scal = scal.at[B:, 2].set(5.0)
    scal = scal.at[B:, 3].set(LBD_DUR / 256.0)

    nsteps = (jnp.max(len_a).astype(jnp.int32) + 7) // 8

    Dr = jnp.concatenate([D, D[:, -1:, :]], axis=1)
    Dt = jnp.transpose(Dr, (1, 0, 2))
    dpad = jnp.zeros((L1 + 2, SUB, LANES), jnp.float32)
    dpad = dpad.at[1:, :B, 1 : L2 + 1].set(Dt)

    ii = jnp.arange(L1 + 2, dtype=jnp.float32)[:, None]
    jj = jnp.arange(LANES, dtype=jnp.float32)[None, :]
    bld = 1.0 - jnp.exp(-((ii - jj) ** 2) / (2.0 * SIGMA**2 * (jj + 1.0)))
    bld = bld.reshape(L1 + 2, 1, LANES)

    t = jnp.arange(BLK, dtype=jnp.float32)[:, None]
    s = jnp.arange(BLK, dtype=jnp.float32)[None, :]
    kp_in = jnp.where(t <= s - 1.0, 1.0, jnp.exp(-(t - s + 1.0)))
    kp_nx = jnp.exp(-(BLK + t - s + 1.0))
    ku_in = jnp.where(t <= s - 1.0, jnp.exp(-(s - t)), jnp.exp(-1.0))
    ku_pv = jnp.exp(-(s + BLK - t))
    kps = jnp.concatenate([kp_in, kp_nx], axis=0).astype(jnp.bfloat16)
    kus = jnp.concatenate([ku_pv, ku_in], axis=0).astype(jnp.bfloat16)

    out = pl.pallas_call(
        _dp_kernel,
        out_shape=jax.ShapeDtypeStruct((SUB, 128), jnp.float32),
        in_specs=[
            pl.BlockSpec(memory_space=pltpu.SMEM),
            pl.BlockSpec(memory_space=pltpu.VMEM),
            pl.BlockSpec(memory_space=pltpu.VMEM),
            pl.BlockSpec(memory_space=pltpu.VMEM),
            pl.BlockSpec(memory_space=pltpu.VMEM),
            pl.BlockSpec(memory_space=pltpu.VMEM),
        ],
    )(nsteps.reshape(1), scal, dpad, bld, kps, kus)
    return out[:B, 0]
